# baseline (speedup 1.0000x reference)
import jax
import jax.numpy as jnp
from jax import lax
from jax.experimental import pallas as pl
from jax.experimental.pallas import tpu as pltpu
from jax.experimental.pallas import tpu_sc as plsc

N = 10000
E = 320000
H = 128
NR = 460

NC = 2
NS = 16
NW = NC * NS
CH = 128

N_PAD = 10240
R_PAD = 512
NPT = N_PAD // NS
RPT = R_PAD // NS

BIG = 256
STEPS1 = 40
PT1 = STEPS1 * BIG
M1 = NW * PT1
STEPS2 = 79
PT2 = STEPS2 * BIG
M2 = NW * PT2
T2 = N + NR

_mesh = plsc.VectorSubcoreMesh(core_axis_name="c", subcore_axis_name="s")


def _wid():
    return lax.axis_index("c") * NS + lax.axis_index("s")


def _sc_pass1(h_hbm, src_hbm, dst_hbm, et_hbm, zrow_hbm,
              s1_out, rs_out,
              gidx, sidx, eidx, rows,
              s1_acc, rs_acc, gsem):
    c = lax.axis_index("c")
    sid = lax.axis_index("s")
    w = _wid()
    base = w * PT1

    pltpu.sync_copy(zrow_hbm, s1_acc.at[pl.ds(sid * NPT, NPT)])
    pltpu.sync_copy(zrow_hbm.at[pl.ds(0, RPT)], rs_acc.at[pl.ds(sid * RPT, RPT)])
    plsc.subcore_barrier()

    def body(i, carry):
        off = base + i * BIG
        pltpu.sync_copy(src_hbm.at[pl.ds(off, BIG)], gidx)
        pltpu.sync_copy(dst_hbm.at[pl.ds(off, BIG)], sidx)
        pltpu.sync_copy(et_hbm.at[pl.ds(off, BIG)], eidx)
        pltpu.async_copy(h_hbm.at[gidx], rows, gsem).wait()
        pltpu.sync_copy(rows, s1_acc.at[sidx], add=True)
        pltpu.sync_copy(rows, rs_acc.at[eidx], add=True)
        return carry

    lax.fori_loop(0, STEPS1, body, 0)
    plsc.subcore_barrier()

    pltpu.sync_copy(s1_acc.at[pl.ds(sid * NPT, NPT)], s1_out.at[c].at[pl.ds(sid * NPT, NPT)])
    pltpu.sync_copy(rs_acc.at[pl.ds(sid * RPT, RPT)], rs_out.at[c].at[pl.ds(sid * RPT, RPT)])


_pass1 = pl.kernel(
    _sc_pass1,
    out_type=(
        jax.ShapeDtypeStruct((NC, N_PAD, H), jnp.float32),
        jax.ShapeDtypeStruct((NC, R_PAD, H), jnp.float32),
    ),
    mesh=_mesh,
    scratch_types=[
        pltpu.VMEM((BIG,), jnp.int32),
        pltpu.VMEM((BIG,), jnp.int32),
        pltpu.VMEM((BIG,), jnp.int32),
        pltpu.VMEM((BIG, H), jnp.float32),
        pltpu.VMEM_SHARED((N_PAD, H), jnp.float32),
        pltpu.VMEM_SHARED((R_PAD, H), jnp.float32),
        pltpu.SemaphoreType.DMA,
    ],
)


def _sc_counts(dst_hbm, et_hbm, zrow_hbm, ones_hbm,
               deg_out, cnt_out,
               sidx, eidx, ones_rows,
               deg_acc, cnt_acc):
    c = lax.axis_index("c")
    sid = lax.axis_index("s")
    w = _wid()
    base = w * PT1

    pltpu.sync_copy(zrow_hbm, deg_acc.at[pl.ds(sid * NPT, NPT)])
    pltpu.sync_copy(zrow_hbm.at[pl.ds(0, RPT)], cnt_acc.at[pl.ds(sid * RPT, RPT)])
    pltpu.sync_copy(ones_hbm, ones_rows)
    plsc.subcore_barrier()

    def body(i, carry):
        off = base + i * BIG
        pltpu.sync_copy(dst_hbm.at[pl.ds(off, BIG)], sidx)
        pltpu.sync_copy(et_hbm.at[pl.ds(off, BIG)], eidx)
        pltpu.sync_copy(ones_rows, deg_acc.at[sidx], add=True)
        pltpu.sync_copy(ones_rows, cnt_acc.at[eidx], add=True)
        return carry

    lax.fori_loop(0, STEPS1, body, 0)
    plsc.subcore_barrier()

    pltpu.sync_copy(deg_acc.at[pl.ds(sid * NPT, NPT)], deg_out.at[c].at[pl.ds(sid * NPT, NPT)])
    pltpu.sync_copy(cnt_acc.at[pl.ds(sid * RPT, RPT)], cnt_out.at[c].at[pl.ds(sid * RPT, RPT)])


_pass_counts = pl.kernel(
    _sc_counts,
    out_type=(
        jax.ShapeDtypeStruct((NC, N_PAD, H), jnp.float32),
        jax.ShapeDtypeStruct((NC, R_PAD, H), jnp.float32),
    ),
    mesh=_mesh,
    scratch_types=[
        pltpu.VMEM((BIG,), jnp.int32),
        pltpu.VMEM((BIG,), jnp.int32),
        pltpu.VMEM((BIG, H), jnp.float32),
        pltpu.VMEM_SHARED((N_PAD, H), jnp.float32),
        pltpu.VMEM_SHARED((R_PAD, H), jnp.float32),
    ],
)


def _make_seg_pass(steps, pt):
    def body_fn(table_hbm, gidx_hbm, sidx_hbm, zrow_hbm,
                s_out,
                gidx, sidx, rows, s_acc, gsem):
        c = lax.axis_index("c")
        sid = lax.axis_index("s")
        w = _wid()
        base = w * pt
        pltpu.sync_copy(zrow_hbm, s_acc.at[pl.ds(sid * NPT, NPT)])
        plsc.subcore_barrier()

        def body(i, carry):
            off = base + i * BIG
            pltpu.sync_copy(gidx_hbm.at[pl.ds(off, BIG)], gidx)
            pltpu.sync_copy(sidx_hbm.at[pl.ds(off, BIG)], sidx)
            pltpu.async_copy(table_hbm.at[gidx], rows, gsem).wait()
            pltpu.sync_copy(rows, s_acc.at[sidx], add=True)
            return carry

        lax.fori_loop(0, steps, body, 0)
        plsc.subcore_barrier()
        pltpu.sync_copy(s_acc.at[pl.ds(sid * NPT, NPT)],
                        s_out.at[c].at[pl.ds(sid * NPT, NPT)])

    return pl.kernel(
        body_fn,
        out_type=jax.ShapeDtypeStruct((NC, N_PAD, H), jnp.float32),
        mesh=_mesh,
        scratch_types=[
            pltpu.VMEM((BIG,), jnp.int32),
            pltpu.VMEM((BIG,), jnp.int32),
            pltpu.VMEM((BIG, H), jnp.float32),
            pltpu.VMEM_SHARED((N_PAD, H), jnp.float32),
            pltpu.SemaphoreType.DMA,
        ],
    )


_pass_rel = _make_seg_pass(STEPS1, PT1)
_pass2 = _make_seg_pass(STEPS2, PT2)


def _mm(x, w):
    return lax.dot_general(x, w, dimension_numbers=(((1,), (0,)), ((), ())),
                           preferred_element_type=jnp.float32)


def _mm_t(x, w):
    return lax.dot_general(x, w, dimension_numbers=(((1,), (1,)), ((), ())),
                           preferred_element_type=jnp.float32)


def _l2n(x):
    n = jnp.sqrt(jnp.sum(x * x, axis=-1, keepdims=True))
    return x / jnp.maximum(n, 1e-12)


def _rrelu(x):
    slope = (1.0 / 8.0 + 1.0 / 3.0) / 2.0
    return jnp.where(x >= 0, x, x * slope)


def _tc_norm_body(x_ref, o_ref):
    o_ref[...] = _l2n(x_ref[...])


def _tc_norm(x):
    return pl.pallas_call(
        _tc_norm_body,
        out_shape=jax.ShapeDtypeStruct((N, H), jnp.float32),
    )(x)


def _tc_gru_body(rs_ref, cnt_ref, er_ref, wih_ref, whh_ref, bih_ref, bhh_ref,
                 w1_ref, w2_ref,
                 h0_ref, t1b_ref, relw2_ref):
    RS = rs_ref[0, :NR] + rs_ref[1, :NR]
    cnt = cnt_ref[0, :NR, 0] + cnt_ref[1, :NR, 0]
    x_input = RS / jnp.maximum(cnt, 1.0)[:, None]
    er = er_ref[...]
    wih = wih_ref[...]
    gi = _mm_t(er, wih[:, :H]) + _mm_t(x_input, wih[:, H:]) + bih_ref[...]
    gh = _mm_t(er, whh_ref[...]) + bhh_ref[...]
    r = jax.nn.sigmoid(gi[:, :H] + gh[:, :H])
    z = jax.nn.sigmoid(gi[:, H:2 * H] + gh[:, H:2 * H])
    nn_ = jnp.tanh(gi[:, 2 * H:] + r * gh[:, 2 * H:])
    h0 = _l2n((1.0 - z) * nn_ + z * er)
    h0_ref[...] = h0
    t1b = jnp.concatenate([_mm(h0, w1_ref[...]),
                           jnp.zeros((R_PAD - NR, H), jnp.float32)], axis=0)
    t1b_ref[...] = t1b
    relw2_ref[...] = _mm(h0, w2_ref[...])


def _tc_gru(rs, cnt, emb_rel, W_ih, W_hh, b_ih, b_hh, W1, W2):
    return pl.pallas_call(
        _tc_gru_body,
        out_shape=(
            jax.ShapeDtypeStruct((NR, H), jnp.float32),
            jax.ShapeDtypeStruct((R_PAD, H), jnp.float32),
            jax.ShapeDtypeStruct((NR, H), jnp.float32),
        ),
    )(rs, cnt, emb_rel, W_ih, W_hh, b_ih, b_hh, W1, W2)


def _tc_layer1_body(s1_ref, sr1_ref, deg_ref, h_ref, w1_ref, wl1_ref, w2_ref,
                    cur1_ref, c1w2_ref):
    deg = deg_ref[0, :N, 0] + deg_ref[1, :N, 0]
    inv = 1.0 / jnp.maximum(deg, 1.0)
    S1 = s1_ref[0, :N] + s1_ref[1, :N]
    SR1 = sr1_ref[0, :N] + sr1_ref[1, :N]
    h = h_ref[...]
    cur1 = _rrelu((_mm(S1, w1_ref[...]) + SR1) * inv[:, None]
                  + _mm(h, wl1_ref[...]))
    cur1_ref[...] = cur1
    c1w2_ref[...] = _mm(cur1, w2_ref[...])


def _tc_layer1(s1, sr1, deg, h, W1, Wl1, W2):
    return pl.pallas_call(
        _tc_layer1_body,
        out_shape=(
            jax.ShapeDtypeStruct((N, H), jnp.float32),
            jax.ShapeDtypeStruct((N, H), jnp.float32),
        ),
    )(s1, sr1, deg, h, W1, Wl1, W2)


def _tc_layer2_body(s2_ref, deg_ref, cur1_ref, wl2_ref, h_ref, tw_ref, tb_ref,
                    out_ref):
    deg = deg_ref[0, :N, 0] + deg_ref[1, :N, 0]
    inv = 1.0 / jnp.maximum(deg, 1.0)
    S2 = s2_ref[0, :N] + s2_ref[1, :N]
    cur1 = cur1_ref[...]
    h = h_ref[...]
    cur2 = _rrelu(S2 * inv[:, None] + _mm(cur1, wl2_ref[...]))
    cur = _l2n(cur2)
    tw = jax.nn.sigmoid(_mm(h, tw_ref[...]) + tb_ref[...])
    out_ref[...] = tw * cur + (1.0 - tw) * h


def _tc_layer2(s2, deg, cur1, Wl2, h, TW, tb):
    return pl.pallas_call(
        _tc_layer2_body,
        out_shape=jax.ShapeDtypeStruct((N, H), jnp.float32),
    )(s2, deg, cur1, Wl2, h, TW, tb)


def kernel(edge_index, edge_type, use_cuda, dynamic_emb, emb_rel, W_ih, W_hh,
           b_ih, b_hh, W_neigh1, W_loop1, W_neigh2, W_loop2,
           time_gate_weight, time_gate_bias):
    src = edge_index[0]
    dst = edge_index[1]
    et = edge_type

    pad1 = M1 - E
    srcp = jnp.concatenate([src, jnp.zeros((pad1,), jnp.int32)])
    dstp = jnp.concatenate([dst, jnp.full((pad1,), N, jnp.int32)])
    etp = jnp.concatenate([et, jnp.full((pad1,), NR, jnp.int32)])

    pad2 = M2 - 2 * E
    gidx2 = jnp.concatenate([src, N + et, jnp.zeros((pad2,), jnp.int32)])
    sidx2 = jnp.concatenate([dst, dst, jnp.full((pad2,), N, jnp.int32)])

    zrow = jnp.zeros((NPT, H), jnp.float32)
    orow = jnp.ones((BIG, H), jnp.float32)

    degp, cntp = _pass_counts(dstp, etp, zrow, orow)

    h = _tc_norm(dynamic_emb)

    s1p, rsp = _pass1(h, srcp, dstp, etp, zrow)

    h_0, t1b, relw2 = _tc_gru(rsp, cntp, emb_rel, W_ih, W_hh, b_ih, b_hh,
                              W_neigh1, W_neigh2)

    sr1p = _pass_rel(t1b, etp, dstp, zrow)

    cur1, c1w2 = _tc_layer1(s1p, sr1p, degp, h, W_neigh1, W_loop1, W_neigh2)

    table2 = jnp.concatenate([c1w2, relw2], axis=0)
    s2p = _pass2(table2, gidx2, sidx2, zrow)

    h_new = _tc_layer2(s2p, degp, cur1, W_loop2, h, time_gate_weight,
                       time_gate_bias)
    return (h_new, h_0)


# speedup vs baseline: 1.2325x; 1.2325x over previous
import functools

import jax
import jax.numpy as jnp
from jax import lax
from jax.experimental import pallas as pl
from jax.experimental.pallas import tpu as pltpu
from jax.experimental.pallas import tpu_sc as plsc

N = 10000
E = 320000
H = 128
NR = 460

NC = 2
NS = 16
NW = NC * NS
CH = 128

N_PAD = 10240
R_PAD = 512
NPT = N_PAD // NS
RPT = R_PAD // NS

CHUNKS1 = 79
PT1 = CHUNKS1 * CH
M1 = NW * PT1
CHUNKS2 = 157
PT2 = CHUNKS2 * CH
M2 = NW * PT2
T2 = N + NR

_mesh = plsc.VectorSubcoreMesh(core_axis_name="c", subcore_axis_name="s")


def _wid():
    return lax.axis_index("c") * NS + lax.axis_index("s")


def _sc_pass1(h_hbm, src_hbm, dst_hbm, et_hbm, zrow_hbm,
              s1_out, rs_out,
              idxg, idxs, idxe, rows,
              s1_acc, rs_acc, gsem):
    c = lax.axis_index("c")
    sid = lax.axis_index("s")
    w = _wid()

    pltpu.sync_copy(zrow_hbm, s1_acc.at[pl.ds(sid * NPT, NPT)])
    pltpu.sync_copy(zrow_hbm.at[pl.ds(0, RPT)], rs_acc.at[pl.ds(sid * RPT, RPT)])
    plsc.subcore_barrier()

    def body(i, carry):
        base = w * PT1 + i * CH
        pltpu.sync_copy(src_hbm.at[pl.ds(base, CH)], idxg)
        pltpu.sync_copy(dst_hbm.at[pl.ds(base, CH)], idxs)
        pltpu.sync_copy(et_hbm.at[pl.ds(base, CH)], idxe)
        pltpu.async_copy(h_hbm.at[idxg], rows, gsem).wait()
        pltpu.sync_copy(rows, s1_acc.at[idxs], add=True)
        pltpu.sync_copy(rows, rs_acc.at[idxe], add=True)
        return carry

    lax.fori_loop(0, CHUNKS1, body, 0)
    plsc.subcore_barrier()

    pltpu.sync_copy(s1_acc.at[pl.ds(sid * NPT, NPT)], s1_out.at[c].at[pl.ds(sid * NPT, NPT)])
    pltpu.sync_copy(rs_acc.at[pl.ds(sid * RPT, RPT)], rs_out.at[c].at[pl.ds(sid * RPT, RPT)])


_pass1 = pl.kernel(
    _sc_pass1,
    out_type=(
        jax.ShapeDtypeStruct((NC, N_PAD, H), jnp.float32),
        jax.ShapeDtypeStruct((NC, R_PAD, H), jnp.float32),
    ),
    mesh=_mesh,
    scratch_types=[
        pltpu.VMEM((CH,), jnp.int32),
        pltpu.VMEM((CH,), jnp.int32),
        pltpu.VMEM((CH,), jnp.int32),
        pltpu.VMEM((CH, H), jnp.float32),
        pltpu.VMEM_SHARED((N_PAD, H), jnp.float32),
        pltpu.VMEM_SHARED((R_PAD, H), jnp.float32),
        pltpu.SemaphoreType.DMA,
    ],
)


def _sc_counts(dst_hbm, et_hbm, zrow_hbm, ones_hbm,
               deg_out, cnt_out,
               idxs, idxe, ones_rows,
               deg_acc, cnt_acc):
    c = lax.axis_index("c")
    sid = lax.axis_index("s")
    w = _wid()

    pltpu.sync_copy(zrow_hbm, deg_acc.at[pl.ds(sid * NPT, NPT)])
    pltpu.sync_copy(zrow_hbm.at[pl.ds(0, RPT)], cnt_acc.at[pl.ds(sid * RPT, RPT)])
    pltpu.sync_copy(ones_hbm, ones_rows)
    plsc.subcore_barrier()

    def body(i, carry):
        base = w * PT1 + i * CH
        pltpu.sync_copy(dst_hbm.at[pl.ds(base, CH)], idxs)
        pltpu.sync_copy(et_hbm.at[pl.ds(base, CH)], idxe)
        pltpu.sync_copy(ones_rows, deg_acc.at[idxs], add=True)
        pltpu.sync_copy(ones_rows, cnt_acc.at[idxe], add=True)
        return carry

    lax.fori_loop(0, CHUNKS1, body, 0)
    plsc.subcore_barrier()

    pltpu.sync_copy(deg_acc.at[pl.ds(sid * NPT, NPT)], deg_out.at[c].at[pl.ds(sid * NPT, NPT)])
    pltpu.sync_copy(cnt_acc.at[pl.ds(sid * RPT, RPT)], cnt_out.at[c].at[pl.ds(sid * RPT, RPT)])


_pass_counts = pl.kernel(
    _sc_counts,
    out_type=(
        jax.ShapeDtypeStruct((NC, N_PAD, H), jnp.float32),
        jax.ShapeDtypeStruct((NC, R_PAD, H), jnp.float32),
    ),
    mesh=_mesh,
    scratch_types=[
        pltpu.VMEM((CH,), jnp.int32),
        pltpu.VMEM((CH,), jnp.int32),
        pltpu.VMEM((CH, H), jnp.float32),
        pltpu.VMEM_SHARED((N_PAD, H), jnp.float32),
        pltpu.VMEM_SHARED((R_PAD, H), jnp.float32),
    ],
)


def _make_seg_pass(table_rows, chunks, pt):
    def body_fn(table_hbm, gidx_hbm, sidx_hbm, zrow_hbm,
                s_out,
                idxg, idxs, rows, s_acc, gsem):
        c = lax.axis_index("c")
        sid = lax.axis_index("s")
        w = _wid()
        pltpu.sync_copy(zrow_hbm, s_acc.at[pl.ds(sid * NPT, NPT)])
        plsc.subcore_barrier()

        def body(i, carry):
            base = w * pt + i * CH
            pltpu.sync_copy(gidx_hbm.at[pl.ds(base, CH)], idxg)
            pltpu.sync_copy(sidx_hbm.at[pl.ds(base, CH)], idxs)
            pltpu.async_copy(table_hbm.at[idxg], rows, gsem).wait()
            pltpu.sync_copy(rows, s_acc.at[idxs], add=True)
            return carry

        lax.fori_loop(0, chunks, body, 0)
        plsc.subcore_barrier()
        pltpu.sync_copy(s_acc.at[pl.ds(sid * NPT, NPT)],
                        s_out.at[c].at[pl.ds(sid * NPT, NPT)])

    return pl.kernel(
        body_fn,
        out_type=jax.ShapeDtypeStruct((NC, N_PAD, H), jnp.float32),
        mesh=_mesh,
        scratch_types=[
            pltpu.VMEM((CH,), jnp.int32),
            pltpu.VMEM((CH,), jnp.int32),
            pltpu.VMEM((CH, H), jnp.float32),
            pltpu.VMEM_SHARED((N_PAD, H), jnp.float32),
            pltpu.SemaphoreType.DMA,
        ],
    )


_pass_rel = _make_seg_pass(R_PAD, CHUNKS1, PT1)
_pass2 = _make_seg_pass(T2, CHUNKS2, PT2)


def _mm(x, w):
    return lax.dot_general(x, w, dimension_numbers=(((1,), (0,)), ((), ())),
                           preferred_element_type=jnp.float32)


def _mm_t(x, w):
    return lax.dot_general(x, w, dimension_numbers=(((1,), (1,)), ((), ())),
                           preferred_element_type=jnp.float32)


def _l2n(x):
    n = jnp.sqrt(jnp.sum(x * x, axis=-1, keepdims=True))
    return x / jnp.maximum(n, 1e-12)


def _rrelu(x):
    slope = (1.0 / 8.0 + 1.0 / 3.0) / 2.0
    return jnp.where(x >= 0, x, x * slope)


def _tc_norm_body(x_ref, o_ref):
    o_ref[...] = _l2n(x_ref[...])


def _tc_norm(x):
    return pl.pallas_call(
        _tc_norm_body,
        out_shape=jax.ShapeDtypeStruct((N, H), jnp.float32),
    )(x)


def _tc_gru_body(rs_ref, cnt_ref, er_ref, wih_ref, whh_ref, bih_ref, bhh_ref,
                 w1_ref, w2_ref,
                 h0_ref, t1b_ref, relw2_ref):
    RS = rs_ref[0, :NR] + rs_ref[1, :NR]
    cnt = cnt_ref[0, :NR, 0] + cnt_ref[1, :NR, 0]
    x_input = RS / jnp.maximum(cnt, 1.0)[:, None]
    er = er_ref[...]
    wih = wih_ref[...]
    gi = _mm_t(er, wih[:, :H]) + _mm_t(x_input, wih[:, H:]) + bih_ref[...]
    gh = _mm_t(er, whh_ref[...]) + bhh_ref[...]
    r = jax.nn.sigmoid(gi[:, :H] + gh[:, :H])
    z = jax.nn.sigmoid(gi[:, H:2 * H] + gh[:, H:2 * H])
    nn_ = jnp.tanh(gi[:, 2 * H:] + r * gh[:, 2 * H:])
    h0 = _l2n((1.0 - z) * nn_ + z * er)
    h0_ref[...] = h0
    t1b = jnp.concatenate([_mm(h0, w1_ref[...]),
                           jnp.zeros((R_PAD - NR, H), jnp.float32)], axis=0)
    t1b_ref[...] = t1b
    relw2_ref[...] = _mm(h0, w2_ref[...])


def _tc_gru(rs, cnt, emb_rel, W_ih, W_hh, b_ih, b_hh, W1, W2):
    return pl.pallas_call(
        _tc_gru_body,
        out_shape=(
            jax.ShapeDtypeStruct((NR, H), jnp.float32),
            jax.ShapeDtypeStruct((R_PAD, H), jnp.float32),
            jax.ShapeDtypeStruct((NR, H), jnp.float32),
        ),
    )(rs, cnt, emb_rel, W_ih, W_hh, b_ih, b_hh, W1, W2)


def _tc_layer1_body(s1_ref, sr1_ref, deg_ref, h_ref, w1_ref, wl1_ref, w2_ref,
                    cur1_ref, c1w2_ref):
    deg = deg_ref[0, :N, 0] + deg_ref[1, :N, 0]
    inv = 1.0 / jnp.maximum(deg, 1.0)
    S1 = s1_ref[0, :N] + s1_ref[1, :N]
    SR1 = sr1_ref[0, :N] + sr1_ref[1, :N]
    h = h_ref[...]
    cur1 = _rrelu((_mm(S1, w1_ref[...]) + SR1) * inv[:, None]
                  + _mm(h, wl1_ref[...]))
    cur1_ref[...] = cur1
    c1w2_ref[...] = _mm(cur1, w2_ref[...])


def _tc_layer1(s1, sr1, deg, h, W1, Wl1, W2):
    return pl.pallas_call(
        _tc_layer1_body,
        out_shape=(
            jax.ShapeDtypeStruct((N, H), jnp.float32),
            jax.ShapeDtypeStruct((N, H), jnp.float32),
        ),
    )(s1, sr1, deg, h, W1, Wl1, W2)


def _tc_layer2_body(s2_ref, deg_ref, cur1_ref, wl2_ref, h_ref, tw_ref, tb_ref,
                    out_ref):
    deg = deg_ref[0, :N, 0] + deg_ref[1, :N, 0]
    inv = 1.0 / jnp.maximum(deg, 1.0)
    S2 = s2_ref[0, :N] + s2_ref[1, :N]
    cur1 = cur1_ref[...]
    h = h_ref[...]
    cur2 = _rrelu(S2 * inv[:, None] + _mm(cur1, wl2_ref[...]))
    cur = _l2n(cur2)
    tw = jax.nn.sigmoid(_mm(h, tw_ref[...]) + tb_ref[...])
    out_ref[...] = tw * cur + (1.0 - tw) * h


def _tc_layer2(s2, deg, cur1, Wl2, h, TW, tb):
    return pl.pallas_call(
        _tc_layer2_body,
        out_shape=jax.ShapeDtypeStruct((N, H), jnp.float32),
    )(s2, deg, cur1, Wl2, h, TW, tb)


def kernel(edge_index, edge_type, use_cuda, dynamic_emb, emb_rel, W_ih, W_hh,
           b_ih, b_hh, W_neigh1, W_loop1, W_neigh2, W_loop2,
           time_gate_weight, time_gate_bias):
    src = edge_index[0]
    dst = edge_index[1]
    et = edge_type

    pad1 = M1 - E
    srcp = jnp.concatenate([src, jnp.zeros((pad1,), jnp.int32)])
    dstp = jnp.concatenate([dst, jnp.full((pad1,), N, jnp.int32)])
    etp = jnp.concatenate([et, jnp.full((pad1,), NR, jnp.int32)])

    pad2 = M2 - 2 * E
    gidx2 = jnp.concatenate([src, N + et, jnp.zeros((pad2,), jnp.int32)])
    sidx2 = jnp.concatenate([dst, dst, jnp.full((pad2,), N, jnp.int32)])

    zrow = jnp.zeros((NPT, H), jnp.float32)
    orow = jnp.ones((CH, H), jnp.float32)

    degp, cntp = _pass_counts(dstp, etp, zrow, orow)

    h = _tc_norm(dynamic_emb)

    s1p, rsp = _pass1(h, srcp, dstp, etp, zrow)

    h_0, t1b, relw2 = _tc_gru(rsp, cntp, emb_rel, W_ih, W_hh, b_ih, b_hh,
                              W_neigh1, W_neigh2)

    sr1p = _pass_rel(t1b, etp, dstp, zrow)

    cur1, c1w2 = _tc_layer1(s1p, sr1p, degp, h, W_neigh1, W_loop1, W_neigh2)

    table2 = jnp.concatenate([c1w2, relw2], axis=0)
    s2p = _pass2(table2, gidx2, sidx2, zrow)

    h_new = _tc_layer2(s2p, degp, cur1, W_loop2, h, time_gate_weight,
                       time_gate_bias)
    return (h_new, h_0)


# speedup vs baseline: 1.2373x; 1.0039x over previous
import jax
import jax.numpy as jnp
from jax import lax
from jax.experimental import pallas as pl
from jax.experimental.pallas import tpu as pltpu
from jax.experimental.pallas import tpu_sc as plsc

N = 10000
E = 320000
H = 128
NR = 460

NC = 2
NS = 16
NW = NC * NS
CH = 128

N_PAD = 10240
R_PAD = 512
NPT = N_PAD // NS
RPT = R_PAD // NS

CHUNKS1 = 79
PT1 = CHUNKS1 * CH
M1 = NW * PT1
CHUNKS2 = 157
PT2 = CHUNKS2 * CH
M2 = NW * PT2
T2 = N + NR

_mesh = plsc.VectorSubcoreMesh(core_axis_name="c", subcore_axis_name="s")


def _wid():
    return lax.axis_index("c") * NS + lax.axis_index("s")


def _sc_pass1(h_hbm, src_hbm, dst_hbm, et_hbm, zrow_hbm,
              s1_out, rs_out,
              idxg, idxs, idxe, rows,
              s1_acc, rs_acc, gsem):
    c = lax.axis_index("c")
    sid = lax.axis_index("s")
    w = _wid()

    pltpu.sync_copy(zrow_hbm, s1_acc.at[pl.ds(sid * NPT, NPT)])
    pltpu.sync_copy(zrow_hbm.at[pl.ds(0, RPT)], rs_acc.at[pl.ds(sid * RPT, RPT)])
    plsc.subcore_barrier()

    def body(i, carry):
        base = w * PT1 + i * CH
        pltpu.sync_copy(src_hbm.at[pl.ds(base, CH)], idxg)
        pltpu.sync_copy(dst_hbm.at[pl.ds(base, CH)], idxs)
        pltpu.sync_copy(et_hbm.at[pl.ds(base, CH)], idxe)
        pltpu.async_copy(h_hbm.at[idxg], rows, gsem).wait()
        pltpu.sync_copy(rows, s1_acc.at[idxs], add=True)
        pltpu.sync_copy(rows, rs_acc.at[idxe], add=True)
        return carry

    lax.fori_loop(0, CHUNKS1, body, 0)
    plsc.subcore_barrier()

    pltpu.sync_copy(s1_acc.at[pl.ds(sid * NPT, NPT)], s1_out.at[c].at[pl.ds(sid * NPT, NPT)])
    pltpu.sync_copy(rs_acc.at[pl.ds(sid * RPT, RPT)], rs_out.at[c].at[pl.ds(sid * RPT, RPT)])


_pass1 = pl.kernel(
    _sc_pass1,
    out_type=(
        jax.ShapeDtypeStruct((NC, N_PAD, H), jnp.float32),
        jax.ShapeDtypeStruct((NC, R_PAD, H), jnp.float32),
    ),
    mesh=_mesh,
    scratch_types=[
        pltpu.VMEM((CH,), jnp.int32),
        pltpu.VMEM((CH,), jnp.int32),
        pltpu.VMEM((CH,), jnp.int32),
        pltpu.VMEM((CH, H), jnp.float32),
        pltpu.VMEM_SHARED((N_PAD, H), jnp.float32),
        pltpu.VMEM_SHARED((R_PAD, H), jnp.float32),
        pltpu.SemaphoreType.DMA,
    ],
)


def _sc_counts(dst_hbm, et_hbm, zrow_hbm, ones_hbm,
               deg_out, cnt_out,
               idxs, idxe, ones_rows,
               deg_acc, cnt_acc):
    c = lax.axis_index("c")
    sid = lax.axis_index("s")
    w = _wid()

    pltpu.sync_copy(zrow_hbm, deg_acc.at[pl.ds(sid * NPT, NPT)])
    pltpu.sync_copy(zrow_hbm.at[pl.ds(0, RPT)], cnt_acc.at[pl.ds(sid * RPT, RPT)])
    pltpu.sync_copy(ones_hbm, ones_rows)
    plsc.subcore_barrier()

    def body(i, carry):
        base = w * PT1 + i * CH
        pltpu.sync_copy(dst_hbm.at[pl.ds(base, CH)], idxs)
        pltpu.sync_copy(et_hbm.at[pl.ds(base, CH)], idxe)
        pltpu.sync_copy(ones_rows, deg_acc.at[idxs], add=True)
        pltpu.sync_copy(ones_rows, cnt_acc.at[idxe], add=True)
        return carry

    lax.fori_loop(0, CHUNKS1, body, 0)
    plsc.subcore_barrier()

    pltpu.sync_copy(deg_acc.at[pl.ds(sid * NPT, NPT)], deg_out.at[c].at[pl.ds(sid * NPT, NPT)])
    pltpu.sync_copy(cnt_acc.at[pl.ds(sid * RPT, RPT)], cnt_out.at[c].at[pl.ds(sid * RPT, RPT)])


_pass_counts = pl.kernel(
    _sc_counts,
    out_type=(
        jax.ShapeDtypeStruct((NC, N_PAD, H), jnp.float32),
        jax.ShapeDtypeStruct((NC, R_PAD, H), jnp.float32),
    ),
    mesh=_mesh,
    scratch_types=[
        pltpu.VMEM((CH,), jnp.int32),
        pltpu.VMEM((CH,), jnp.int32),
        pltpu.VMEM((CH, H), jnp.float32),
        pltpu.VMEM_SHARED((N_PAD, H), jnp.float32),
        pltpu.VMEM_SHARED((R_PAD, H), jnp.float32),
    ],
)


def _make_seg_pass(table_rows, chunks, pt):
    def body_fn(table_hbm, gidx_hbm, sidx_hbm, zrow_hbm,
                s_out,
                idxg, idxs, rows, s_acc, gsem):
        c = lax.axis_index("c")
        sid = lax.axis_index("s")
        w = _wid()
        pltpu.sync_copy(zrow_hbm, s_acc.at[pl.ds(sid * NPT, NPT)])
        plsc.subcore_barrier()

        def body(i, carry):
            base = w * pt + i * CH
            pltpu.sync_copy(gidx_hbm.at[pl.ds(base, CH)], idxg)
            pltpu.sync_copy(sidx_hbm.at[pl.ds(base, CH)], idxs)
            pltpu.async_copy(table_hbm.at[idxg], rows, gsem).wait()
            pltpu.sync_copy(rows, s_acc.at[idxs], add=True)
            return carry

        lax.fori_loop(0, chunks, body, 0)
        plsc.subcore_barrier()
        pltpu.sync_copy(s_acc.at[pl.ds(sid * NPT, NPT)],
                        s_out.at[c].at[pl.ds(sid * NPT, NPT)])

    return pl.kernel(
        body_fn,
        out_type=jax.ShapeDtypeStruct((NC, N_PAD, H), jnp.float32),
        mesh=_mesh,
        scratch_types=[
            pltpu.VMEM((CH,), jnp.int32),
            pltpu.VMEM((CH,), jnp.int32),
            pltpu.VMEM((CH, H), jnp.float32),
            pltpu.VMEM_SHARED((N_PAD, H), jnp.float32),
            pltpu.SemaphoreType.DMA,
        ],
    )


_pass_rel = _make_seg_pass(R_PAD, CHUNKS1, PT1)
_pass2 = _make_seg_pass(T2, CHUNKS2, PT2)


def _mm(x, w):
    return lax.dot_general(x, w, dimension_numbers=(((1,), (0,)), ((), ())),
                           preferred_element_type=jnp.float32)


def _mm_t(x, w):
    return lax.dot_general(x, w, dimension_numbers=(((1,), (1,)), ((), ())),
                           preferred_element_type=jnp.float32)


def _l2n(x):
    n = jnp.sqrt(jnp.sum(x * x, axis=-1, keepdims=True))
    return x / jnp.maximum(n, 1e-12)


def _rrelu(x):
    slope = (1.0 / 8.0 + 1.0 / 3.0) / 2.0
    return jnp.where(x >= 0, x, x * slope)


def _tc_norm_body(x_ref, o_ref):
    o_ref[...] = _l2n(x_ref[...])


def _tc_norm(x):
    return pl.pallas_call(
        _tc_norm_body,
        out_shape=jax.ShapeDtypeStruct((N, H), jnp.float32),
    )(x)


def _tc_gru_body(rs_ref, cnt_ref, er_ref, wih_ref, whh_ref, bih_ref, bhh_ref,
                 w1_ref, w2_ref,
                 h0_ref, t1b_ref, relw2_ref):
    RS = rs_ref[0, :NR] + rs_ref[1, :NR]
    cnt = cnt_ref[0, :NR, 0] + cnt_ref[1, :NR, 0]
    x_input = RS / jnp.maximum(cnt, 1.0)[:, None]
    er = er_ref[...]
    wih = wih_ref[...]
    gi = _mm_t(er, wih[:, :H]) + _mm_t(x_input, wih[:, H:]) + bih_ref[...]
    gh = _mm_t(er, whh_ref[...]) + bhh_ref[...]
    r = jax.nn.sigmoid(gi[:, :H] + gh[:, :H])
    z = jax.nn.sigmoid(gi[:, H:2 * H] + gh[:, H:2 * H])
    nn_ = jnp.tanh(gi[:, 2 * H:] + r * gh[:, 2 * H:])
    h0 = _l2n((1.0 - z) * nn_ + z * er)
    h0_ref[...] = h0
    t1b = jnp.concatenate([_mm(h0, w1_ref[...]),
                           jnp.zeros((R_PAD - NR, H), jnp.float32)], axis=0)
    t1b_ref[...] = t1b
    relw2_ref[...] = _mm(h0, w2_ref[...])


def _tc_gru(rs, cnt, emb_rel, W_ih, W_hh, b_ih, b_hh, W1, W2):
    return pl.pallas_call(
        _tc_gru_body,
        out_shape=(
            jax.ShapeDtypeStruct((NR, H), jnp.float32),
            jax.ShapeDtypeStruct((R_PAD, H), jnp.float32),
            jax.ShapeDtypeStruct((NR, H), jnp.float32),
        ),
    )(rs, cnt, emb_rel, W_ih, W_hh, b_ih, b_hh, W1, W2)


def _tc_layer1_body(s1_ref, sr1_ref, deg_ref, h_ref, w1_ref, wl1_ref, w2_ref,
                    cur1_ref, c1w2_ref):
    deg = deg_ref[0, :N, 0] + deg_ref[1, :N, 0]
    inv = 1.0 / jnp.maximum(deg, 1.0)
    S1 = s1_ref[0, :N] + s1_ref[1, :N]
    SR1 = sr1_ref[0, :N] + sr1_ref[1, :N]
    h = h_ref[...]
    cur1 = _rrelu((_mm(S1, w1_ref[...]) + SR1) * inv[:, None]
                  + _mm(h, wl1_ref[...]))
    cur1_ref[...] = cur1
    c1w2_ref[...] = _mm(cur1, w2_ref[...])


def _tc_layer1(s1, sr1, deg, h, W1, Wl1, W2):
    return pl.pallas_call(
        _tc_layer1_body,
        out_shape=(
            jax.ShapeDtypeStruct((N, H), jnp.float32),
            jax.ShapeDtypeStruct((N, H), jnp.float32),
        ),
    )(s1, sr1, deg, h, W1, Wl1, W2)


def _tc_layer2_body(s2_ref, deg_ref, cur1_ref, wl2_ref, h_ref, tw_ref, tb_ref,
                    out_ref):
    deg = deg_ref[0, :N, 0] + deg_ref[1, :N, 0]
    inv = 1.0 / jnp.maximum(deg, 1.0)
    S2 = s2_ref[0, :N] + s2_ref[1, :N]
    cur1 = cur1_ref[...]
    h = h_ref[...]
    cur2 = _rrelu(S2 * inv[:, None] + _mm(cur1, wl2_ref[...]))
    cur = _l2n(cur2)
    tw = jax.nn.sigmoid(_mm(h, tw_ref[...]) + tb_ref[...])
    out_ref[...] = tw * cur + (1.0 - tw) * h


def _tc_layer2(s2, deg, cur1, Wl2, h, TW, tb):
    return pl.pallas_call(
        _tc_layer2_body,
        out_shape=jax.ShapeDtypeStruct((N, H), jnp.float32),
    )(s2, deg, cur1, Wl2, h, TW, tb)


def kernel(edge_index, edge_type, use_cuda, dynamic_emb, emb_rel, W_ih, W_hh,
           b_ih, b_hh, W_neigh1, W_loop1, W_neigh2, W_loop2,
           time_gate_weight, time_gate_bias):
    src = edge_index[0]
    dst = edge_index[1]
    et = edge_type

    pad1 = M1 - E
    srcp = jnp.concatenate([src, jnp.zeros((pad1,), jnp.int32)])
    dstp = jnp.concatenate([dst, jnp.full((pad1,), N, jnp.int32)])
    etp = jnp.concatenate([et, jnp.full((pad1,), NR, jnp.int32)])

    pad2 = M2 - 2 * E
    gidx2 = jnp.concatenate([src, N + et, jnp.zeros((pad2,), jnp.int32)])
    sidx2 = jnp.concatenate([dst, dst, jnp.full((pad2,), N, jnp.int32)])

    zrow = jnp.zeros((NPT, H), jnp.float32)
    orow = jnp.ones((CH, H), jnp.float32)

    degp, cntp = _pass_counts(dstp, etp, zrow, orow)

    h = _tc_norm(dynamic_emb)

    s1p, rsp = _pass1(h, srcp, dstp, etp, zrow)

    h_0, t1b, relw2 = _tc_gru(rsp, cntp, emb_rel, W_ih, W_hh, b_ih, b_hh,
                              W_neigh1, W_neigh2)

    sr1p = _pass_rel(t1b, etp, dstp, zrow)

    cur1, c1w2 = _tc_layer1(s1p, sr1p, degp, h, W_neigh1, W_loop1, W_neigh2)

    table2 = jnp.concatenate([c1w2, relw2], axis=0)
    s2p = _pass2(table2, gidx2, sidx2, zrow)

    h_new = _tc_layer2(s2p, degp, cur1, W_loop2, h, time_gate_weight,
                       time_gate_bias)
    return (h_new, h_0)
